# Initial kernel scaffold; baseline (speedup 1.0000x reference)
import jax
import jax.numpy as jnp
from jax.experimental import pallas as pl


def kernel(h, edge_index, edge_attr, W):
    raise NotImplementedError("write your pallas kernel here")


# speedup vs baseline: 2.6561x; 2.6561x over previous
import functools

import jax
import jax.numpy as jnp
from jax import lax
from jax.experimental import pallas as pl
from jax.experimental.pallas import tpu as pltpu
from jax.experimental.pallas import tpu_sc as plsc

_NC = 2
_NS = 16
_C = 80
_EW = 16


def _sc_agg(NP, DH, NCH, NPT):
  mesh = plsc.VectorSubcoreMesh(core_axis_name="c", subcore_axis_name="s")

  @functools.partial(
      pl.kernel,
      mesh=mesh,
      compiler_params=pltpu.CompilerParams(use_tc_tiling_on_sc=False),
      out_type=[
          jax.ShapeDtypeStruct((_NC, NP, DH), jnp.float32),
          jax.ShapeDtypeStruct((_NC, NP, _EW), jnp.float32),
      ],
      scratch_types=[
          pltpu.VMEM((NCH, _C), jnp.int32),
          pltpu.VMEM((NCH, _C), jnp.int32),
          pltpu.VMEM((_C, DH), jnp.float32),
          pltpu.VMEM((_C, _EW), jnp.float32),
          pltpu.VMEM_SHARED((NP, DH), jnp.float32),
          pltpu.VMEM_SHARED((NP, _EW), jnp.float32),
          pltpu.SemaphoreType.DMA,
      ],
  )
  def sc_agg(src_hbm, dst_hbm, ea_hbm, h_hbm, zh_hbm, ze_hbm,
             outh_hbm, oute_hbm,
             idx_s, idx_d, rows_v, ea_v, acc_h, acc_e, sem):
    c = lax.axis_index("c")
    s = lax.axis_index("s")
    r0 = s * NPT
    pltpu.sync_copy(zh_hbm, acc_h.at[pl.ds(r0, NPT)])
    pltpu.sync_copy(ze_hbm, acc_e.at[pl.ds(r0, NPT)])
    pltpu.sync_copy(src_hbm.at[s], idx_s)
    pltpu.sync_copy(dst_hbm.at[s], idx_d)
    plsc.subcore_barrier()

    def body(j, carry):
      pltpu.async_copy(h_hbm.at[c].at[idx_s.at[j]], rows_v, sem).wait()
      pltpu.sync_copy(ea_hbm.at[c].at[s].at[j], ea_v)
      pltpu.sync_copy(rows_v, acc_h.at[idx_d.at[j]], add=True)
      pltpu.sync_copy(ea_v, acc_e.at[idx_d.at[j]], add=True)
      return carry

    lax.fori_loop(0, NCH, body, 0)
    plsc.subcore_barrier()
    pltpu.sync_copy(acc_h.at[pl.ds(r0, NPT)], outh_hbm.at[c].at[pl.ds(r0, NPT)])
    pltpu.sync_copy(acc_e.at[pl.ds(r0, NPT)], oute_hbm.at[c].at[pl.ds(r0, NPT)])

  return sc_agg


def _tc_final(h, ph, pe, W, elast, N, D, DE, DOUT, DH):
  B = 400

  def body(h_ref, ph_ref, pe_ref, w_ref, el_ref, o_ref):
    hb = h_ref[...]
    ah0 = ph_ref[0]
    ah1 = ph_ref[1]
    ae = pe_ref[0]
    cnt = pe_ref[1][:, 0:1]
    inv = 1.0 / jnp.maximum(cnt, 1.0)
    iso = cnt == 0.0
    w1 = w_ref[:D]
    w2 = w_ref[D:2 * D]
    w3 = w_ref[2 * D:]
    dot = functools.partial(jnp.dot, preferred_element_type=jnp.float32)
    base = dot(hb, w1)
    agg = (dot(ah0, w2[:DH]) + dot(ah1, w2[DH:]) + dot(ae, w3)) * inv
    iso_mm = dot(hb, w2) + dot(el_ref[0:1, :], w3)
    out = base + jnp.where(iso, iso_mm, agg)
    o_ref[...] = jnp.where(out > 0.0, out, jnp.exp(out) - 1.0)

  return pl.pallas_call(
      body,
      grid=(N // B,),
      in_specs=[
          pl.BlockSpec((B, D), lambda i: (i, 0)),
          pl.BlockSpec((_NC, B, DH), lambda i: (0, i, 0)),
          pl.BlockSpec((_NC, B, _EW), lambda i: (0, i, 0)),
          pl.BlockSpec((2 * D + DE, DOUT), lambda i: (0, 0)),
          pl.BlockSpec((8, DE), lambda i: (0, 0)),
      ],
      out_specs=pl.BlockSpec((B, DOUT), lambda i: (i, 0)),
      out_shape=jax.ShapeDtypeStruct((N, DOUT), jnp.float32),
  )(h, ph, pe, W, elast)


def kernel(h, edge_index, edge_attr, W):
  N, D = h.shape
  E = edge_index.shape[1]
  DE = edge_attr.shape[1]
  DOUT = W.shape[1]
  DH = D // _NC

  ept = E // _NS
  nch = ept // _C
  npt = -(-(N // _NS) // 8) * 8
  np_ = npt * _NS

  dst = edge_index[0]
  src = edge_index[1]

  src_r = src.reshape(_NS, nch, _C)
  dst_r = dst.reshape(_NS, nch, _C)
  ones_rows = jnp.concatenate(
      [jnp.ones((E, 1), jnp.float32),
       jnp.zeros((E, _EW - 1), jnp.float32)], axis=1)
  ea_r = jnp.stack([edge_attr, ones_rows]).reshape(_NC, _NS, nch, _C, _EW)
  h_halves = jnp.stack([h[:, :DH], h[:, DH:]])
  zh = jnp.zeros((npt, DH), jnp.float32)
  ze = jnp.zeros((npt, _EW), jnp.float32)

  ph, pe = _sc_agg(np_, DH, nch, npt)(src_r, dst_r, ea_r, h_halves, zh, ze)

  elast = jnp.broadcast_to(edge_attr[-1], (8, DE))
  return _tc_final(h, ph, pe, W, elast, N, D, DE, DOUT, DH)


# speedup vs baseline: 7.2446x; 2.7275x over previous
import functools

import jax
import jax.numpy as jnp
from jax import lax
from jax.experimental import pallas as pl
from jax.experimental.pallas import tpu as pltpu
from jax.experimental.pallas import tpu_sc as plsc

_NC = 2
_NS = 16
_C = 80
_EW = 16


def _sc_agg(NP, DH, NCH, NPT, DE):
  mesh = plsc.VectorSubcoreMesh(core_axis_name="c", subcore_axis_name="s")
  G = NCH // 2

  @functools.partial(
      pl.kernel,
      mesh=mesh,
      compiler_params=pltpu.CompilerParams(use_tc_tiling_on_sc=False),
      out_type=[
          jax.ShapeDtypeStruct((_NC, NP, DH), jnp.float32),
          jax.ShapeDtypeStruct((_NC, NP, _EW), jnp.float32),
      ],
      scratch_types=[
          pltpu.VMEM((NCH, _C), jnp.int32),
          pltpu.VMEM((NCH, _C), jnp.int32),
          pltpu.VMEM((_C, DH), jnp.float32),
          pltpu.VMEM((_C, DH), jnp.float32),
          pltpu.VMEM((_C, _EW), jnp.float32),
          pltpu.VMEM((_C, _EW), jnp.float32),
          pltpu.VMEM_SHARED((NP, DH), jnp.float32),
          pltpu.VMEM_SHARED((NP, _EW), jnp.float32),
          pltpu.SemaphoreType.DMA,
          pltpu.SemaphoreType.DMA,
          pltpu.SemaphoreType.DMA,
          pltpu.SemaphoreType.DMA,
      ],
  )
  def sc_agg(src_hbm, dst_hbm, ea_hbm, ones_hbm, h_hbm, zh_hbm, ze_hbm,
             outh_hbm, oute_hbm,
             idx_s, idx_d, rows_v0, rows_v1, ea_v0, ea_v1,
             acc_h, acc_e, semA, semB, semC, semD):
    c = lax.axis_index("c")
    s = lax.axis_index("s")
    r0 = s * NPT
    hc = h_hbm.at[c]
    pltpu.sync_copy(zh_hbm, acc_h.at[pl.ds(r0, NPT)])
    pltpu.sync_copy(ze_hbm, acc_e.at[pl.ds(r0, NPT)])
    pltpu.sync_copy(src_hbm.at[s], idx_s)
    pltpu.sync_copy(dst_hbm.at[s], idx_d)
    plsc.subcore_barrier()

    pltpu.async_copy(hc.at[idx_s.at[0]], rows_v0, semA)

    @pl.when(c == 0)
    def _():
      pltpu.async_copy(ea_hbm.at[s].at[0], ea_v0, semC)

    @pl.when(c == 1)
    def _():
      pltpu.sync_copy(ones_hbm, ea_v0)
      pltpu.sync_copy(ones_hbm, ea_v1)

    zh80 = zh_hbm.at[pl.ds(0, _C)]
    ze80 = ze_hbm.at[pl.ds(0, _C)]

    def body(g, carry):
      j0 = 2 * g
      j1 = j0 + 1
      pltpu.async_copy(hc.at[idx_s.at[j1]], rows_v1, semB)

      @pl.when(c == 0)
      def _():
        pltpu.async_copy(ea_hbm.at[s].at[j1], ea_v1, semD)

      pltpu.make_async_copy(zh80, rows_v0, semA).wait()
      pltpu.sync_copy(rows_v0, acc_h.at[idx_d.at[j0]], add=True)

      @pl.when(c == 0)
      def _():
        pltpu.make_async_copy(ze80, ea_v0, semC).wait()

      pltpu.sync_copy(ea_v0, acc_e.at[idx_d.at[j0]], add=True)

      @pl.when(g < G - 1)
      def _():
        pltpu.async_copy(hc.at[idx_s.at[j0 + 2]], rows_v0, semA)

      @pl.when(jnp.logical_and(g < G - 1, c == 0))
      def _():
        pltpu.async_copy(ea_hbm.at[s].at[j0 + 2], ea_v0, semC)

      pltpu.make_async_copy(zh80, rows_v1, semB).wait()
      pltpu.sync_copy(rows_v1, acc_h.at[idx_d.at[j1]], add=True)

      @pl.when(c == 0)
      def _():
        pltpu.make_async_copy(ze80, ea_v1, semD).wait()

      pltpu.sync_copy(ea_v1, acc_e.at[idx_d.at[j1]], add=True)
      return carry

    lax.fori_loop(0, G, body, 0)
    plsc.subcore_barrier()
    pltpu.sync_copy(acc_h.at[pl.ds(r0, NPT)], outh_hbm.at[c].at[pl.ds(r0, NPT)])
    pltpu.sync_copy(acc_e.at[pl.ds(r0, NPT)], oute_hbm.at[c].at[pl.ds(r0, NPT)])

  return sc_agg


def _tc_final(h, ph, pe, W, elast, N, D, DE, DOUT, DH):
  B = 400

  def body(h_ref, ph_ref, pe_ref, w_ref, el_ref, o_ref):
    hb = h_ref[...]
    ah0 = ph_ref[0]
    ah1 = ph_ref[1]
    ae = pe_ref[0]
    cnt = pe_ref[1][:, 0:1]
    inv = 1.0 / jnp.maximum(cnt, 1.0)
    iso = cnt == 0.0
    w1 = w_ref[:D]
    w2 = w_ref[D:2 * D]
    w3 = w_ref[2 * D:]
    dot = functools.partial(jnp.dot, preferred_element_type=jnp.float32)
    base = dot(hb, w1)
    agg = (dot(ah0, w2[:DH]) + dot(ah1, w2[DH:]) + dot(ae, w3)) * inv
    iso_mm = dot(hb, w2) + dot(el_ref[0:1, :], w3)
    out = base + jnp.where(iso, iso_mm, agg)
    o_ref[...] = jnp.where(out > 0.0, out, jnp.exp(out) - 1.0)

  return pl.pallas_call(
      body,
      grid=(N // B,),
      in_specs=[
          pl.BlockSpec((B, D), lambda i: (i, 0)),
          pl.BlockSpec((_NC, B, DH), lambda i: (0, i, 0)),
          pl.BlockSpec((_NC, B, _EW), lambda i: (0, i, 0)),
          pl.BlockSpec((2 * D + DE, DOUT), lambda i: (0, 0)),
          pl.BlockSpec((8, DE), lambda i: (0, 0)),
      ],
      out_specs=pl.BlockSpec((B, DOUT), lambda i: (i, 0)),
      out_shape=jax.ShapeDtypeStruct((N, DOUT), jnp.float32),
  )(h, ph, pe, W, elast)


def kernel(h, edge_index, edge_attr, W):
  N, D = h.shape
  E = edge_index.shape[1]
  DE = edge_attr.shape[1]
  DOUT = W.shape[1]
  DH = D // _NC

  ept = E // _NS
  nch = ept // _C
  npt = -(-(N // _NS) // 8) * 8
  np_ = npt * _NS

  dst = edge_index[0]
  src = edge_index[1]
  src_r = src.reshape(_NS, nch, _C)
  dst_r = dst.reshape(_NS, nch, _C)
  ea_r = edge_attr.reshape(_NS, nch, _C, _EW)
  ones_c = jnp.concatenate(
      [jnp.ones((_C, 1), jnp.float32),
       jnp.zeros((_C, _EW - 1), jnp.float32)], axis=1)
  h_halves = jnp.stack([h[:, :DH], h[:, DH:]])
  zh = jnp.zeros((npt, DH), jnp.float32)
  ze = jnp.zeros((npt, _EW), jnp.float32)

  ph, pe = _sc_agg(np_, DH, nch, npt, DE)(
      src_r, dst_r, ea_r, ones_c, h_halves, zh, ze)

  elast = jnp.broadcast_to(edge_attr[-1], (8, DE))
  return _tc_final(h, ph, pe, W, elast, N, D, DE, DOUT, DH)


# speedup vs baseline: 7.7464x; 1.0693x over previous
import functools

import jax
import jax.numpy as jnp
from jax import lax
from jax.experimental import pallas as pl
from jax.experimental.pallas import tpu as pltpu
from jax.experimental.pallas import tpu_sc as plsc

_NC = 2
_NS = 16
_C = 80
_EW = 16


def _sc_agg(NP, DH, NCH, NPT, DE):
  mesh = plsc.VectorSubcoreMesh(core_axis_name="c", subcore_axis_name="s")
  G = NCH // 2

  @functools.partial(
      pl.kernel,
      mesh=mesh,
      compiler_params=pltpu.CompilerParams(use_tc_tiling_on_sc=False),
      out_type=[
          jax.ShapeDtypeStruct((_NC, NP, DH), jnp.float32),
          jax.ShapeDtypeStruct((_NC, NP, _EW), jnp.float32),
      ],
      scratch_types=[
          pltpu.VMEM((NCH, _C), jnp.int32),
          pltpu.VMEM((NCH, _C), jnp.int32),
          pltpu.VMEM((_C, DH), jnp.float32),
          pltpu.VMEM((_C, DH), jnp.float32),
          pltpu.VMEM((_C, _EW), jnp.float32),
          pltpu.VMEM((_C, _EW), jnp.float32),
          pltpu.VMEM_SHARED((NP, DH), jnp.float32),
          pltpu.VMEM_SHARED((NP, _EW), jnp.float32),
          pltpu.SemaphoreType.DMA,
          pltpu.SemaphoreType.DMA,
          pltpu.SemaphoreType.DMA,
          pltpu.SemaphoreType.DMA,
      ],
  )
  def sc_agg(src_hbm, dst_hbm, ea_hbm, ones_hbm, h_hbm, zh_hbm, ze_hbm,
             outh_hbm, oute_hbm,
             idx_s, idx_d, rows_v0, rows_v1, ea_v0, ea_v1,
             acc_h, acc_e, semA, semB, semC, semD):
    c = lax.axis_index("c")
    s = lax.axis_index("s")
    r0 = s * NPT
    hc = h_hbm
    pltpu.sync_copy(zh_hbm, acc_h.at[pl.ds(r0, NPT)])
    pltpu.sync_copy(ze_hbm, acc_e.at[pl.ds(r0, NPT)])
    pltpu.sync_copy(src_hbm.at[s], idx_s)
    pltpu.sync_copy(dst_hbm.at[s], idx_d)
    def fix(j, carry):
      for t in range(_C // 16):
        sl = (j, pl.ds(t * 16, 16))
        idx_s[sl] = idx_s[sl] * 2 + c
      return carry

    lax.fori_loop(0, NCH, fix, 0)
    plsc.subcore_barrier()

    pltpu.async_copy(hc.at[idx_s.at[0]], rows_v0, semA)

    @pl.when(c == 0)
    def _():
      pltpu.async_copy(ea_hbm.at[s].at[0], ea_v0, semC)

    @pl.when(c == 1)
    def _():
      pltpu.sync_copy(ones_hbm, ea_v0)
      pltpu.sync_copy(ones_hbm, ea_v1)

    zh80 = zh_hbm.at[pl.ds(0, _C)]
    ze80 = ze_hbm.at[pl.ds(0, _C)]

    def body(g, carry):
      j0 = 2 * g
      j1 = j0 + 1
      pltpu.async_copy(hc.at[idx_s.at[j1]], rows_v1, semB)

      @pl.when(c == 0)
      def _():
        pltpu.async_copy(ea_hbm.at[s].at[j1], ea_v1, semD)

      pltpu.make_async_copy(zh80, rows_v0, semA).wait()
      pltpu.sync_copy(rows_v0, acc_h.at[idx_d.at[j0]], add=True)

      @pl.when(c == 0)
      def _():
        pltpu.make_async_copy(ze80, ea_v0, semC).wait()

      pltpu.sync_copy(ea_v0, acc_e.at[idx_d.at[j0]], add=True)

      @pl.when(g < G - 1)
      def _():
        pltpu.async_copy(hc.at[idx_s.at[j0 + 2]], rows_v0, semA)

      @pl.when(jnp.logical_and(g < G - 1, c == 0))
      def _():
        pltpu.async_copy(ea_hbm.at[s].at[j0 + 2], ea_v0, semC)

      pltpu.make_async_copy(zh80, rows_v1, semB).wait()
      pltpu.sync_copy(rows_v1, acc_h.at[idx_d.at[j1]], add=True)

      @pl.when(c == 0)
      def _():
        pltpu.make_async_copy(ze80, ea_v1, semD).wait()

      pltpu.sync_copy(ea_v1, acc_e.at[idx_d.at[j1]], add=True)
      return carry

    lax.fori_loop(0, G, body, 0)
    plsc.subcore_barrier()
    pltpu.sync_copy(acc_h.at[pl.ds(r0, NPT)], outh_hbm.at[c].at[pl.ds(r0, NPT)])
    pltpu.sync_copy(acc_e.at[pl.ds(r0, NPT)], oute_hbm.at[c].at[pl.ds(r0, NPT)])

  return sc_agg


def _tc_final(h, ph, pe, W, elast, N, D, DE, DOUT, DH):
  B = 400

  def body(h_ref, ph_ref, pe_ref, w_ref, el_ref, o_ref):
    hb = h_ref[...]
    ah0 = ph_ref[0]
    ah1 = ph_ref[1]
    ae = pe_ref[0]
    cnt = pe_ref[1][:, 0:1]
    inv = 1.0 / jnp.maximum(cnt, 1.0)
    iso = cnt == 0.0
    w1 = w_ref[:D]
    w2 = w_ref[D:2 * D]
    w3 = w_ref[2 * D:]
    dot = functools.partial(jnp.dot, preferred_element_type=jnp.float32)
    base = dot(hb, w1)
    agg = (dot(ah0, w2[:DH]) + dot(ah1, w2[DH:]) + dot(ae, w3)) * inv
    iso_mm = dot(hb, w2) + dot(el_ref[0:1, :], w3)
    out = base + jnp.where(iso, iso_mm, agg)
    o_ref[...] = jnp.where(out > 0.0, out, jnp.exp(out) - 1.0)

  return pl.pallas_call(
      body,
      grid=(N // B,),
      in_specs=[
          pl.BlockSpec((B, D), lambda i: (i, 0)),
          pl.BlockSpec((_NC, B, DH), lambda i: (0, i, 0)),
          pl.BlockSpec((_NC, B, _EW), lambda i: (0, i, 0)),
          pl.BlockSpec((2 * D + DE, DOUT), lambda i: (0, 0)),
          pl.BlockSpec((8, DE), lambda i: (0, 0)),
      ],
      out_specs=pl.BlockSpec((B, DOUT), lambda i: (i, 0)),
      out_shape=jax.ShapeDtypeStruct((N, DOUT), jnp.float32),
  )(h, ph, pe, W, elast)


def kernel(h, edge_index, edge_attr, W):
  N, D = h.shape
  E = edge_index.shape[1]
  DE = edge_attr.shape[1]
  DOUT = W.shape[1]
  DH = D // _NC

  ept = E // _NS
  nch = ept // _C
  npt = -(-(N // _NS) // 8) * 8
  np_ = npt * _NS

  dst = edge_index[0]
  src = edge_index[1]
  src_r = src.reshape(_NS, nch, _C)
  dst_r = dst.reshape(_NS, nch, _C)
  ea_r = edge_attr.reshape(_NS, nch, _C, _EW)
  ones_c = jnp.concatenate(
      [jnp.ones((_C, 1), jnp.float32),
       jnp.zeros((_C, _EW - 1), jnp.float32)], axis=1)
  h_view = h.reshape(_NC * N, DH)
  zh = jnp.zeros((npt, DH), jnp.float32)
  ze = jnp.zeros((npt, _EW), jnp.float32)

  ph, pe = _sc_agg(np_, DH, nch, npt, DE)(
      src_r, dst_r, ea_r, ones_c, h_view, zh, ze)

  elast = jnp.broadcast_to(edge_attr[-1], (8, DE))
  return _tc_final(h, ph, pe, W, elast, N, D, DE, DOUT, DH)
